# baseline (speedup 1.0000x reference)
import functools
import jax
import jax.numpy as jnp
from jax import lax
from jax.experimental import pallas as pl
from jax.experimental.pallas import tpu as pltpu
from jax.experimental.pallas import tpu_sc as plsc

_B, _S, _D = 16, 2048, 512
_NV, _H = 32, 512
_T = _S + _NV
_HALF = _T // 2
_CH = 80
_ZCH = 40
_LANES = 16


def _mlp_body(emb_ref, w1_ref, b1_ref, w2_ref, b2_ref, out_ref):
    h = jnp.tanh(
        jnp.dot(emb_ref[...], w1_ref[...],
                preferred_element_type=jnp.float32) + b1_ref[...])
    out_ref[...] = jnp.dot(
        h, w2_ref[...], preferred_element_type=jnp.float32) + b2_ref[...]


def _virtual_rows(embed_table, W1, b1, W2, b2):
    return pl.pallas_call(
        _mlp_body,
        out_shape=jax.ShapeDtypeStruct((_NV, _D), jnp.float32),
    )(embed_table, W1, b1.reshape(1, _H), W2, b2.reshape(1, _D))


def _sc_body(inp_hbm, seq_hbm, virt_hbm, zeros_hbm, out_hbm,
             buf2, vbuf, zbuf, seqv, semA, semB, semZ, semOutA, semOutB):
    c = lax.axis_index("c")
    s = lax.axis_index("s")
    wid = s * 2 + c
    b = wid % _B
    half = wid // _B
    row0 = half * _HALF
    r1 = row0 + _HALF

    @pl.when(s == 0)
    def _init_zshared():
        pltpu.sync_copy(zeros_hbm, zbuf)

    pltpu.sync_copy(seq_hbm, seqv.at[pl.ds(0, 16)])
    pltpu.sync_copy(virt_hbm, vbuf)
    plsc.subcore_barrier()

    ln = seqv[pl.ds(b, 16)][0]

    copy_rows = jnp.clip(ln - row0, 0, _HALF)
    n_full = copy_rows // _CH

    z0 = jnp.clip(ln + _NV, row0, r1)
    zsu = row0 + ((z0 - row0 + _ZCH - 1) // _ZCH) * _ZCH

    zrows = r1 - zsu
    zbig = (zrows // 640) * 640
    zrem = zrows - zbig

    zq = zrem // _ZCH

    def _zpiece_args():
        args = []
        o = zsu
        args.append((zbig > 0, o, 640))
        o = o + zbig
        for m in (8, 4, 2, 1):
            take = (zq & m) * _ZCH
            args.append((take > 0, o, m * _ZCH))
            o = o + take
        return args

    def _zdo(start):
        for cond, o, z in _zpiece_args():
            @pl.when(cond)
            def _f(o=o, z=z):
                cpy = pltpu.make_async_copy(
                    zbuf.at[pl.ds(0, z)],
                    out_hbm.at[b, pl.ds(pl.multiple_of(o, 8), z), :], semZ)
                if start:
                    cpy.start()
                else:
                    cpy.wait()
    _zdo(True)

    def _src(k):
        ro = pl.multiple_of(row0 + k * _CH, 8)
        return inp_hbm.at[b, pl.ds(ro, _CH), :]

    def _dst(k):
        ro = pl.multiple_of(row0 + k * _CH, 8)
        return out_hbm.at[b, pl.ds(ro, _CH), :]

    @pl.when(n_full > 0)
    def _prologue0():
        pltpu.make_async_copy(_src(0), buf2.at[0], semA).start()

    @pl.when(n_full > 1)
    def _prologue1():
        pltpu.make_async_copy(_src(1), buf2.at[1], semB).start()

    def _cpair(p_, carry):
        k0 = 2 * p_
        k1 = k0 + 1

        @pl.when(k0 > 0)
        def _wout0():
            pltpu.make_async_copy(buf2.at[0], _dst(k0 - 2), semOutA).wait()

        @pl.when(k0 > 0)
        def _start_in0():
            pltpu.make_async_copy(_src(k0), buf2.at[0], semA).start()

        pltpu.make_async_copy(_src(k0), buf2.at[0], semA).wait()
        pltpu.make_async_copy(buf2.at[0], _dst(k0), semOutA).start()

        @pl.when(k1 < n_full)
        def _slot1():
            @pl.when(k1 > 1)
            def _wout1():
                pltpu.make_async_copy(buf2.at[1], _dst(k1 - 2),
                                      semOutB).wait()

            @pl.when(k1 > 1)
            def _start_in1():
                pltpu.make_async_copy(_src(k1), buf2.at[1], semB).start()

            pltpu.make_async_copy(_src(k1), buf2.at[1], semB).wait()
            pltpu.make_async_copy(buf2.at[1], _dst(k1), semOutB).start()
        return carry
    lax.fori_loop(0, (n_full + 1) // 2, _cpair, 0)

    @pl.when(n_full > 0)
    def _drain_out0():
        klast0 = ((n_full - 1) // 2) * 2
        pltpu.make_async_copy(buf2.at[0], _dst(klast0), semOutA).wait()

    @pl.when(n_full > 1)
    def _drain_out1():
        klast1 = ((n_full - 2) // 2) * 2 + 1
        pltpu.make_async_copy(buf2.at[1], _dst(klast1), semOutB).wait()

    owner = jnp.logical_and(ln >= row0, ln < r1)

    @pl.when(owner)
    def _owner_work():
        len8 = (ln // 8) * 8
        pstart = pl.multiple_of(jnp.minimum(len8, _T - _CH), 8)
        p = ln - pstart
        off = row0 + n_full * _CH
        rem8 = pstart - off

        @pl.when(rem8 > 0)
        def _remainder():
            src0 = pl.multiple_of(jnp.minimum(off, _S - _CH), 8)
            delta = off - src0
            pltpu.sync_copy(inp_hbm.at[b, pl.ds(src0, _CH), :], buf2.at[0])
            o = off
            d = delta
            for z in (64, 32, 16, 8):
                take = rem8 & z

                @pl.when(take > 0)
                def _piece(o=o, d=d, z=z):
                    pltpu.sync_copy(
                        buf2.at[0, pl.ds(pl.multiple_of(d, 8), z)],
                        out_hbm.at[b, pl.ds(pl.multiple_of(o, 8), z), :])
                o = o + take
                d = d + take

        pltpu.sync_copy(inp_hbm.at[b, pl.ds(pstart, 48), :],
                        buf2.at[1, pl.ds(0, 48)])

        def _vrow(j, carry):
            for l in range(_D // _LANES):
                buf2[1, p + j, pl.ds(l * _LANES, _LANES)] = (
                    vbuf[j, pl.ds(l * _LANES, _LANES)])
            return carry
        lax.fori_loop(0, _NV, _vrow, 0)

        zero16 = jnp.zeros((_LANES,), jnp.float32)

        def _zrow(j, carry):
            for l in range(_D // _LANES):
                buf2[1, p + _NV + j, pl.ds(l * _LANES, _LANES)] = zero16
            return carry
        lax.fori_loop(0, _CH - _NV - p, _zrow, 0)

        pltpu.sync_copy(buf2.at[1], out_hbm.at[b, pl.ds(pstart, _CH), :])

    _zdo(False)


@functools.partial(
    pl.kernel,
    out_type=jax.ShapeDtypeStruct((_B, _T, _D), jnp.float32),
    mesh=plsc.VectorSubcoreMesh(core_axis_name="c", subcore_axis_name="s"),
    scratch_types=[
        pltpu.VMEM((2, _CH, _D), jnp.float32),
        pltpu.VMEM((_NV, _D), jnp.float32),
        pltpu.VMEM_SHARED((640, _D), jnp.float32),
        pltpu.VMEM((48,), jnp.int32),
        pltpu.SemaphoreType.DMA,
        pltpu.SemaphoreType.DMA,
        pltpu.SemaphoreType.DMA,
        pltpu.SemaphoreType.DMA,
        pltpu.SemaphoreType.DMA,
    ],
)
def _sc_assemble(inp_hbm, seq_hbm, virt_hbm, zeros_hbm, out_hbm,
                 buf2, vbuf, zbuf, seqv, semA, semB, semZ, semOutA, semOutB):
    _sc_body(inp_hbm, seq_hbm, virt_hbm, zeros_hbm, out_hbm,
             buf2, vbuf, zbuf, seqv, semA, semB, semZ, semOutA, semOutB)


def kernel(inputs, seq_len, embed_table, W1, b1, W2, b2):
    seq_len = seq_len.astype(jnp.int32)
    virtual = jnp.zeros((_NV, _D), jnp.float32)
    zeros = jnp.zeros((640, _D), jnp.float32)
    out = _sc_assemble(inputs, seq_len, virtual, zeros)
    return out, seq_len + _NV


# speedup vs baseline: 1.0151x; 1.0151x over previous
import functools
import jax
import jax.numpy as jnp
from jax import lax
from jax.experimental import pallas as pl
from jax.experimental.pallas import tpu as pltpu
from jax.experimental.pallas import tpu_sc as plsc

_B, _S, _D = 16, 2048, 512
_NV, _H = 32, 512
_T = _S + _NV
_HALF = _T // 2
_CH = 80
_ZCH = 40
_LANES = 16


def _mlp_body(emb_ref, w1_ref, b1_ref, w2_ref, b2_ref, out_ref):
    h = jnp.tanh(
        jnp.dot(emb_ref[...], w1_ref[...],
                preferred_element_type=jnp.float32) + b1_ref[...])
    out_ref[...] = jnp.dot(
        h, w2_ref[...], preferred_element_type=jnp.float32) + b2_ref[...]


def _virtual_rows(embed_table, W1, b1, W2, b2):
    return pl.pallas_call(
        _mlp_body,
        out_shape=jax.ShapeDtypeStruct((_NV, _D), jnp.float32),
    )(embed_table, W1, b1.reshape(1, _H), W2, b2.reshape(1, _D))


def _sc_body(inp_hbm, seq_hbm, virt_hbm, zeros_hbm, out_hbm,
             buf2, vbuf, zbuf, seqv, semA, semB, semZ, semOutA, semOutB):
    c = lax.axis_index("c")
    s = lax.axis_index("s")
    wid = s * 2 + c
    b = wid % _B
    half = wid // _B
    row0 = half * _HALF
    r1 = row0 + _HALF

    @pl.when(s == 0)
    def _init_zshared():
        pltpu.sync_copy(zeros_hbm, zbuf)

    pltpu.sync_copy(seq_hbm, seqv.at[pl.ds(0, 16)])
    pltpu.sync_copy(virt_hbm, vbuf)
    plsc.subcore_barrier()

    ln = seqv[pl.ds(b, 16)][0]

    copy_rows = jnp.clip(ln - row0, 0, _HALF)
    n_full = copy_rows // _CH

    z0 = jnp.clip(ln + _NV, row0, r1)
    zsu = row0 + ((z0 - row0 + _ZCH - 1) // _ZCH) * _ZCH

    zrows = r1 - zsu
    zbig = (zrows // 640) * 640
    zrem = zrows - zbig

    zq = zrem // _ZCH

    def _zpiece_args():
        args = []
        o = zsu
        args.append((zbig > 0, o, 640))
        o = o + zbig
        for m in (8, 4, 2, 1):
            take = (zq & m) * _ZCH
            args.append((take > 0, o, m * _ZCH))
            o = o + take
        return args

    def _zdo(start):
        for cond, o, z in _zpiece_args():
            @pl.when(cond)
            def _f(o=o, z=z):
                cpy = pltpu.make_async_copy(
                    zbuf.at[pl.ds(0, z)],
                    out_hbm.at[b, pl.ds(pl.multiple_of(o, 8), z), :], semZ)
                if start:
                    cpy.start()
                else:
                    cpy.wait()
    _zdo(True)

    def _src(k):
        ro = pl.multiple_of(row0 + k * _CH, 8)
        return inp_hbm.at[b, pl.ds(ro, _CH), :]

    def _dst(k):
        ro = pl.multiple_of(row0 + k * _CH, 8)
        return out_hbm.at[b, pl.ds(ro, _CH), :]

    @pl.when(n_full > 0)
    def _prologue0():
        pltpu.make_async_copy(_src(0), buf2.at[0], semA).start()

    @pl.when(n_full > 1)
    def _prologue1():
        pltpu.make_async_copy(_src(1), buf2.at[1], semB).start()

    def _cpair(p_, carry):
        k0 = 2 * p_
        k1 = k0 + 1

        @pl.when(k0 > 0)
        def _wout0():
            pltpu.make_async_copy(buf2.at[0], _dst(k0 - 2), semOutA).wait()

        @pl.when(k0 > 0)
        def _start_in0():
            pltpu.make_async_copy(_src(k0), buf2.at[0], semA).start()

        pltpu.make_async_copy(_src(k0), buf2.at[0], semA).wait()
        pltpu.make_async_copy(buf2.at[0], _dst(k0), semOutA).start()

        @pl.when(k1 < n_full)
        def _slot1():
            @pl.when(k1 > 1)
            def _wout1():
                pltpu.make_async_copy(buf2.at[1], _dst(k1 - 2),
                                      semOutB).wait()

            @pl.when(k1 > 1)
            def _start_in1():
                pltpu.make_async_copy(_src(k1), buf2.at[1], semB).start()

            pltpu.make_async_copy(_src(k1), buf2.at[1], semB).wait()
            pltpu.make_async_copy(buf2.at[1], _dst(k1), semOutB).start()
        return carry
    lax.fori_loop(0, (n_full + 1) // 2, _cpair, 0)

    @pl.when(n_full > 0)
    def _drain_out0():
        klast0 = ((n_full - 1) // 2) * 2
        pltpu.make_async_copy(buf2.at[0], _dst(klast0), semOutA).wait()

    @pl.when(n_full > 1)
    def _drain_out1():
        klast1 = ((n_full - 2) // 2) * 2 + 1
        pltpu.make_async_copy(buf2.at[1], _dst(klast1), semOutB).wait()

    owner = jnp.logical_and(ln >= row0, ln < r1)

    @pl.when(owner)
    def _owner_work():
        len8 = (ln // 8) * 8
        pstart = pl.multiple_of(jnp.minimum(len8, _T - _CH), 8)
        p = ln - pstart
        off = row0 + n_full * _CH
        rem8 = pstart - off

        @pl.when(rem8 > 0)
        def _remainder():
            src0 = pl.multiple_of(jnp.minimum(off, _S - _CH), 8)
            delta = off - src0
            pltpu.sync_copy(inp_hbm.at[b, pl.ds(src0, _CH), :], buf2.at[0])
            o = off
            d = delta
            for z in (64, 32, 16, 8):
                take = rem8 & z

                @pl.when(take > 0)
                def _piece(o=o, d=d, z=z):
                    pltpu.sync_copy(
                        buf2.at[0, pl.ds(pl.multiple_of(d, 8), z)],
                        out_hbm.at[b, pl.ds(pl.multiple_of(o, 8), z), :])
                o = o + take
                d = d + take

        pltpu.sync_copy(inp_hbm.at[b, pl.ds(pstart, 48), :],
                        buf2.at[1, pl.ds(0, 48)])

        def _vrow(j, carry):
            for l in range(_D // _LANES):
                buf2[1, p + j, pl.ds(l * _LANES, _LANES)] = (
                    vbuf[j, pl.ds(l * _LANES, _LANES)])
            return carry
        lax.fori_loop(0, _NV, _vrow, 0)

        zero16 = jnp.zeros((_LANES,), jnp.float32)

        def _zrow(j, carry):
            for l in range(_D // _LANES):
                buf2[1, p + _NV + j, pl.ds(l * _LANES, _LANES)] = zero16
            return carry
        lax.fori_loop(0, _CH - _NV - p, _zrow, 0)

        pltpu.sync_copy(buf2.at[1], out_hbm.at[b, pl.ds(pstart, _CH), :])

    _zdo(False)


@functools.partial(
    pl.kernel,
    out_type=jax.ShapeDtypeStruct((_B, _T, _D), jnp.float32),
    mesh=plsc.VectorSubcoreMesh(core_axis_name="c", subcore_axis_name="s"),
    scratch_types=[
        pltpu.VMEM((2, _CH, _D), jnp.float32),
        pltpu.VMEM((_NV, _D), jnp.float32),
        pltpu.VMEM_SHARED((640, _D), jnp.float32),
        pltpu.VMEM((48,), jnp.int32),
        pltpu.SemaphoreType.DMA,
        pltpu.SemaphoreType.DMA,
        pltpu.SemaphoreType.DMA,
        pltpu.SemaphoreType.DMA,
        pltpu.SemaphoreType.DMA,
    ],
)
def _sc_assemble(inp_hbm, seq_hbm, virt_hbm, zeros_hbm, out_hbm,
                 buf2, vbuf, zbuf, seqv, semA, semB, semZ, semOutA, semOutB):
    _sc_body(inp_hbm, seq_hbm, virt_hbm, zeros_hbm, out_hbm,
             buf2, vbuf, zbuf, seqv, semA, semB, semZ, semOutA, semOutB)


def kernel(inputs, seq_len, embed_table, W1, b1, W2, b2):
    seq_len = seq_len.astype(jnp.int32)
    virtual = _virtual_rows(embed_table, W1, b1, W2, b2)
    zeros = jnp.zeros((640, _D), jnp.float32)
    out = _sc_assemble(inputs, seq_len, virtual, zeros)
    return out, seq_len + _NV


# speedup vs baseline: 1.0181x; 1.0030x over previous
import functools
import jax
import jax.numpy as jnp
from jax import lax
from jax.experimental import pallas as pl
from jax.experimental.pallas import tpu as pltpu
from jax.experimental.pallas import tpu_sc as plsc

_B, _S, _D = 16, 2048, 512
_NV, _H = 32, 512
_T = _S + _NV
_HALF = _T // 2
_CH = 80
_PCH = 80
_ZCH = 40
_LANES = 16


def _mlp_body(emb_ref, w1_ref, b1_ref, w2_ref, b2_ref, out_ref):
    h = jnp.tanh(
        jnp.dot(emb_ref[...], w1_ref[...],
                preferred_element_type=jnp.float32) + b1_ref[...])
    out_ref[...] = jnp.dot(
        h, w2_ref[...], preferred_element_type=jnp.float32) + b2_ref[...]


def _virtual_rows(embed_table, W1, b1, W2, b2):
    return pl.pallas_call(
        _mlp_body,
        out_shape=jax.ShapeDtypeStruct((_NV, _D), jnp.float32),
    )(embed_table, W1, b1.reshape(1, _H), W2, b2.reshape(1, _D))


def _sc_body(inp_hbm, seq_hbm, virt_hbm, zeros_hbm, out_hbm,
             buf2, vbuf, zbuf, seqv, semA, semB, semZ, semOutA, semOutB):
    c = lax.axis_index("c")
    s = lax.axis_index("s")
    wid = s * 2 + c
    b = wid % _B
    half = wid // _B
    row0 = half * _HALF
    r1 = row0 + _HALF

    @pl.when(s == 0)
    def _init_zshared():
        pltpu.sync_copy(zeros_hbm, zbuf)

    pltpu.sync_copy(seq_hbm, seqv.at[pl.ds(0, 16)])
    plsc.subcore_barrier()

    ln = seqv[pl.ds(b, 16)][0]

    copy_rows = jnp.clip(ln - row0, 0, _HALF)
    n_full = copy_rows // _CH

    z0 = jnp.clip(ln + _NV, row0, r1)
    zsu = row0 + ((z0 - row0 + _ZCH - 1) // _ZCH) * _ZCH

    zrows = r1 - zsu
    zbig = (zrows // 640) * 640
    zrem = zrows - zbig

    zq = zrem // _ZCH

    def _zpiece_args():
        args = []
        o = zsu
        args.append((zbig > 0, o, 640))
        o = o + zbig
        for m in (8, 4, 2, 1):
            take = (zq & m) * _ZCH
            args.append((take > 0, o, m * _ZCH))
            o = o + take
        return args

    def _zdo(start):
        for cond, o, z in _zpiece_args():
            @pl.when(cond)
            def _f(o=o, z=z):
                cpy = pltpu.make_async_copy(
                    zbuf.at[pl.ds(0, z)],
                    out_hbm.at[b, pl.ds(pl.multiple_of(o, 8), z), :], semZ)
                if start:
                    cpy.start()
                else:
                    cpy.wait()
    _zdo(True)

    def _src(k):
        ro = pl.multiple_of(row0 + k * _CH, 8)
        return inp_hbm.at[b, pl.ds(ro, _CH), :]

    def _dst(k):
        ro = pl.multiple_of(row0 + k * _CH, 8)
        return out_hbm.at[b, pl.ds(ro, _CH), :]

    @pl.when(n_full > 0)
    def _prologue0():
        pltpu.make_async_copy(_src(0), buf2.at[0], semA).start()

    @pl.when(n_full > 1)
    def _prologue1():
        pltpu.make_async_copy(_src(1), buf2.at[1], semB).start()

    def _cpair(p_, carry):
        k0 = 2 * p_
        k1 = k0 + 1

        @pl.when(k0 > 0)
        def _wout0():
            pltpu.make_async_copy(buf2.at[0], _dst(k0 - 2), semOutA).wait()

        @pl.when(k0 > 0)
        def _start_in0():
            pltpu.make_async_copy(_src(k0), buf2.at[0], semA).start()

        pltpu.make_async_copy(_src(k0), buf2.at[0], semA).wait()
        pltpu.make_async_copy(buf2.at[0], _dst(k0), semOutA).start()

        @pl.when(k1 < n_full)
        def _slot1():
            @pl.when(k1 > 1)
            def _wout1():
                pltpu.make_async_copy(buf2.at[1], _dst(k1 - 2),
                                      semOutB).wait()

            @pl.when(k1 > 1)
            def _start_in1():
                pltpu.make_async_copy(_src(k1), buf2.at[1], semB).start()

            pltpu.make_async_copy(_src(k1), buf2.at[1], semB).wait()
            pltpu.make_async_copy(buf2.at[1], _dst(k1), semOutB).start()
        return carry
    lax.fori_loop(0, (n_full + 1) // 2, _cpair, 0)

    @pl.when(n_full > 0)
    def _drain_out0():
        klast0 = ((n_full - 1) // 2) * 2
        pltpu.make_async_copy(buf2.at[0], _dst(klast0), semOutA).wait()

    @pl.when(n_full > 1)
    def _drain_out1():
        klast1 = ((n_full - 2) // 2) * 2 + 1
        pltpu.make_async_copy(buf2.at[1], _dst(klast1), semOutB).wait()

    owner = jnp.logical_and(ln >= row0, ln < r1)

    @pl.when(owner)
    def _owner_work():
        pltpu.sync_copy(virt_hbm, vbuf)
        len8 = (ln // 8) * 8
        pstart = pl.multiple_of(jnp.minimum(len8, _T - _PCH), 8)
        p = ln - pstart
        off = row0 + n_full * _CH
        rem8 = pstart - off

        @pl.when(rem8 > 0)
        def _remainder():
            src0 = pl.multiple_of(jnp.minimum(off, _S - _CH), 8)
            delta = off - src0
            pltpu.sync_copy(inp_hbm.at[b, pl.ds(src0, _CH), :], buf2.at[0])
            o = off
            d = delta
            for z in (64, 32, 16, 8):
                take = rem8 & z

                @pl.when(take > 0)
                def _piece(o=o, d=d, z=z):
                    pltpu.sync_copy(
                        buf2.at[0, pl.ds(pl.multiple_of(d, 8), z)],
                        out_hbm.at[b, pl.ds(pl.multiple_of(o, 8), z), :])
                o = o + take
                d = d + take

        pltpu.sync_copy(inp_hbm.at[b, pl.ds(pstart, 48), :],
                        buf2.at[1, pl.ds(0, 48)])

        def _vrow(j, carry):
            for l in range(_D // _LANES):
                buf2[1, p + j, pl.ds(l * _LANES, _LANES)] = (
                    vbuf[j, pl.ds(l * _LANES, _LANES)])
            return carry
        lax.fori_loop(0, _NV, _vrow, 0)

        zero16 = jnp.zeros((_LANES,), jnp.float32)

        def _zrow(j, carry):
            for l in range(_D // _LANES):
                buf2[1, p + _NV + j, pl.ds(l * _LANES, _LANES)] = zero16
            return carry
        lax.fori_loop(0, _PCH - _NV - p, _zrow, 0)

        pltpu.sync_copy(buf2.at[1, pl.ds(0, _PCH)],
                        out_hbm.at[b, pl.ds(pstart, _PCH), :])

    _zdo(False)


@functools.partial(
    pl.kernel,
    out_type=jax.ShapeDtypeStruct((_B, _T, _D), jnp.float32),
    mesh=plsc.VectorSubcoreMesh(core_axis_name="c", subcore_axis_name="s"),
    scratch_types=[
        pltpu.VMEM((2, _CH, _D), jnp.float32),
        pltpu.VMEM((_NV, _D), jnp.float32),
        pltpu.VMEM_SHARED((640, _D), jnp.float32),
        pltpu.VMEM((48,), jnp.int32),
        pltpu.SemaphoreType.DMA,
        pltpu.SemaphoreType.DMA,
        pltpu.SemaphoreType.DMA,
        pltpu.SemaphoreType.DMA,
        pltpu.SemaphoreType.DMA,
    ],
)
def _sc_assemble(inp_hbm, seq_hbm, virt_hbm, zeros_hbm, out_hbm,
                 buf2, vbuf, zbuf, seqv, semA, semB, semZ, semOutA, semOutB):
    _sc_body(inp_hbm, seq_hbm, virt_hbm, zeros_hbm, out_hbm,
             buf2, vbuf, zbuf, seqv, semA, semB, semZ, semOutA, semOutB)


def kernel(inputs, seq_len, embed_table, W1, b1, W2, b2):
    seq_len = seq_len.astype(jnp.int32)
    virtual = _virtual_rows(embed_table, W1, b1, W2, b2)
    zeros = jnp.zeros((640, _D), jnp.float32)
    out = _sc_assemble(inputs, seq_len, virtual, zeros)
    return out, seq_len + _NV
